# baseline (speedup 1.0000x reference)
import functools

import jax
import jax.numpy as jnp
from jax import lax
from jax.experimental import pallas as pl
from jax.experimental.pallas import tpu as pltpu
from jax.experimental.pallas import tpu_sc as plsc

VOCAB = 1000
EMBED = 64
B = 4096
L = 200
NC = 2
NS = 16
NW = NC * NS
FLAT = B * L
SUB = L
CPT = B // NW
NBUF = 2
VPR = EMBED // 16
NT = SUB // 16


@functools.cache
def _sc_kernel():
    mesh = plsc.VectorSubcoreMesh(core_axis_name="c", subcore_axis_name="s")

    scratch = [
        pltpu.VMEM((VOCAB, EMBED), jnp.float32),
        pltpu.VMEM((SUB * VPR, 16), jnp.float32),
        pltpu.VMEM((CPT * SUB,), jnp.int32),
    ]
    scratch += [pltpu.VMEM((SUB * VPR, 16), jnp.float32) for _ in range(NBUF)]
    scratch += [pltpu.SemaphoreType.DMA for _ in range(NBUF)]

    @functools.partial(
        pl.kernel,
        mesh=mesh,
        out_type=jax.ShapeDtypeStruct((FLAT * VPR, 16), jnp.float32),
        compiler_params=pltpu.CompilerParams(
            use_tc_tiling_on_sc=False, needs_layout_passes=False),
        scratch_types=scratch,
    )
    def k(x_hbm, ww_hbm, wp_hbm, out_hbm, *refs):
        ww_v, pos_v, x_v = refs[0], refs[1], refs[2]
        bufs = refs[3:3 + NBUF]
        wsems = refs[3 + NBUF:3 + 2 * NBUF]

        wid = lax.axis_index("s") * NC + lax.axis_index("c")
        out_base = wid * CPT * SUB * VPR
        pltpu.sync_copy(ww_hbm, ww_v)
        pltpu.sync_copy(wp_hbm, pos_v)
        pltpu.sync_copy(x_hbm.at[pl.ds(wid * CPT * SUB, CPT * SUB)], x_v)

        def drain_w(sem, buf):
            pltpu.make_async_copy(out_hbm.at[pl.ds(0, SUB * VPR)], buf, sem).wait()

        def do_group(buf, iv, rows, js):
            for j in js:
                xr = iv[j]
                row = rows + j
                for q in range(VPR):
                    buf[row * VPR + q] = (
                        ww_v[xr, pl.ds(q * 16, 16)] + pos_v[row * VPR + q]
                    )

        def body(m, carry):
            for b in range(NBUF):
                kk = NBUF * m + b

                @pl.when(kk >= NBUF)
                def _():
                    drain_w(wsems[b], bufs[b])

                buf = bufs[b]
                kbase = kk * SUB

                def grp(t, c2):
                    iv = x_v[pl.ds(kbase + 16 * t, 16)]
                    for j in range(16):
                        xr = iv[j]
                        row = 16 * t + j
                        for q in range(VPR):
                            buf[row * VPR + q] = (
                                ww_v[xr, pl.ds(q * 16, 16)]
                                + pos_v[row * VPR + q]
                            )
                    return c2

                lax.fori_loop(0, NT, grp, 0)

                iv_t = x_v[pl.ds(kbase + SUB - 16, 16)]
                do_group(buf, iv_t, SUB - 16, range(8, 16))

                pltpu.async_copy(
                    buf,
                    out_hbm.at[pl.ds(out_base + kk * SUB * VPR, SUB * VPR)],
                    wsems[b])
            return carry

        lax.fori_loop(0, CPT // NBUF, body, 0)
        for b in range(NBUF):
            drain_w(wsems[b], bufs[b])

    return k


@jax.jit
def kernel(x, W_words, W_pos):
    x2 = x.reshape(FLAT).astype(jnp.int32)
    wp = W_pos[:L].reshape(SUB * VPR, 16)
    out = _sc_kernel()(x2, W_words, wp)
    return out.reshape(B, L, EMBED)


# speedup vs baseline: 1.2321x; 1.2321x over previous
import functools

import jax
import jax.numpy as jnp
from jax import lax
from jax.experimental import pallas as pl
from jax.experimental.pallas import tpu as pltpu
from jax.experimental.pallas import tpu_sc as plsc

VOCAB = 1000
EMBED = 64
B = 4096
L = 200
NC = 2
NS = 16
NW = NC * NS
FLAT = B * L
SUB = L
CPT = B // NW
NBUF = 2
VPR = EMBED // 16
NT = SUB // 16


@functools.cache
def _sc_kernel():
    mesh = plsc.VectorSubcoreMesh(core_axis_name="c", subcore_axis_name="s")

    scratch = [
        pltpu.VMEM((VOCAB, EMBED), jnp.float32),
        pltpu.VMEM((SUB * VPR, 16), jnp.float32),
        pltpu.VMEM((CPT * SUB,), jnp.int32),
    ]
    scratch += [pltpu.VMEM((SUB * VPR, 16), jnp.float32) for _ in range(NBUF)]
    scratch += [pltpu.SemaphoreType.DMA for _ in range(NBUF)]

    @functools.partial(
        pl.kernel,
        mesh=mesh,
        out_type=jax.ShapeDtypeStruct((FLAT * VPR, 16), jnp.float32),
        compiler_params=pltpu.CompilerParams(
            use_tc_tiling_on_sc=False, needs_layout_passes=False),
        scratch_types=scratch,
    )
    def k(x_hbm, ww_hbm, wp_hbm, out_hbm, *refs):
        ww_v, pos_v, x_v = refs[0], refs[1], refs[2]
        bufs = refs[3:3 + NBUF]
        wsems = refs[3 + NBUF:3 + 2 * NBUF]

        wid = lax.axis_index("s") * NC + lax.axis_index("c")
        out_base = wid * CPT * SUB * VPR
        pltpu.sync_copy(ww_hbm, ww_v)
        pltpu.sync_copy(wp_hbm, pos_v)
        pltpu.sync_copy(x_hbm.at[pl.ds(wid * CPT * SUB, CPT * SUB)], x_v)

        def drain_w(sem, buf):
            pltpu.make_async_copy(out_hbm.at[pl.ds(0, SUB * VPR)], buf, sem).wait()

        def do_group(buf, iv, rows, js):
            for j in js:
                xr = iv[j]
                row = rows + j
                for q in range(VPR):
                    buf[row * VPR + q] = (
                        ww_v[xr, pl.ds(q * 16, 16)] + pos_v[row * VPR + q]
                    )

        def body(m, carry):
            for b in range(NBUF):
                kk = NBUF * m + b

                @pl.when(kk >= NBUF)
                def _():
                    drain_w(wsems[b], bufs[b])

                buf = bufs[b]
                kbase = kk * SUB

                @plsc.parallel_loop(0, NT, unroll=4)
                def _(t):
                    iv = x_v[pl.ds(kbase + 16 * t, 16)]
                    for j in range(16):
                        xr = iv[j]
                        row = 16 * t + j
                        for q in range(VPR):
                            buf[row * VPR + q] = (
                                ww_v[xr, pl.ds(q * 16, 16)]
                                + pos_v[row * VPR + q]
                            )

                iv_t = x_v[pl.ds(kbase + SUB - 16, 16)]
                do_group(buf, iv_t, SUB - 16, range(8, 16))

                pltpu.async_copy(
                    buf,
                    out_hbm.at[pl.ds(out_base + kk * SUB * VPR, SUB * VPR)],
                    wsems[b])
            return carry

        lax.fori_loop(0, CPT // NBUF, body, 0)
        for b in range(NBUF):
            drain_w(wsems[b], bufs[b])

    return k


@jax.jit
def kernel(x, W_words, W_pos):
    x2 = x.reshape(FLAT).astype(jnp.int32)
    wp = W_pos[:L].reshape(SUB * VPR, 16)
    out = _sc_kernel()(x2, W_words, wp)
    return out.reshape(B, L, EMBED)


# speedup vs baseline: 1.2586x; 1.0216x over previous
import functools

import jax
import jax.numpy as jnp
from jax import lax
from jax.experimental import pallas as pl
from jax.experimental.pallas import tpu as pltpu
from jax.experimental.pallas import tpu_sc as plsc

VOCAB = 1000
EMBED = 64
B = 4096
L = 200
NC = 2
NS = 16
NW = NC * NS
FLAT = B * L
SUB = L
CPT = B // NW
NBUF = 2
VPR = EMBED // 16
NT = SUB // 16


@functools.cache
def _sc_kernel():
    mesh = plsc.VectorSubcoreMesh(core_axis_name="c", subcore_axis_name="s")

    scratch = [
        pltpu.VMEM((VOCAB, EMBED), jnp.float32),
        pltpu.VMEM((SUB * VPR, 16), jnp.float32),
        pltpu.VMEM((CPT * SUB,), jnp.int32),
    ]
    scratch += [pltpu.VMEM((SUB * VPR, 16), jnp.float32) for _ in range(NBUF)]
    scratch += [pltpu.SemaphoreType.DMA for _ in range(NBUF)]

    @functools.partial(
        pl.kernel,
        mesh=mesh,
        out_type=jax.ShapeDtypeStruct((FLAT * VPR, 16), jnp.float32),
        compiler_params=pltpu.CompilerParams(
            use_tc_tiling_on_sc=False, needs_layout_passes=False),
        scratch_types=scratch,
    )
    def k(x_hbm, ww_hbm, wp_hbm, out_hbm, *refs):
        ww_v, pos_v, x_v = refs[0], refs[1], refs[2]
        bufs = refs[3:3 + NBUF]
        wsems = refs[3 + NBUF:3 + 2 * NBUF]

        wid = lax.axis_index("s") * NC + lax.axis_index("c")
        out_base = wid * CPT * SUB * VPR
        pltpu.sync_copy(ww_hbm, ww_v)
        pltpu.sync_copy(wp_hbm, pos_v)
        pltpu.sync_copy(x_hbm.at[pl.ds(wid * CPT * SUB, CPT * SUB)], x_v)

        def drain_w(sem, buf):
            pltpu.make_async_copy(out_hbm.at[pl.ds(0, SUB * VPR)], buf, sem).wait()

        def do_group(buf, iv, rows, js):
            for j in js:
                xr = iv[j]
                row = rows + j
                for q in range(VPR):
                    buf[row * VPR + q] = (
                        ww_v[xr, pl.ds(q * 16, 16)] + pos_v[row * VPR + q]
                    )

        def body(m, carry):
            for b in range(NBUF):
                kk = NBUF * m + b

                @pl.when(kk >= NBUF)
                def _():
                    drain_w(wsems[b], bufs[b])

                buf = bufs[b]
                kbase = kk * SUB

                @plsc.parallel_loop(0, NT, unroll=2)
                def _(t):
                    iv = x_v[pl.ds(kbase + 16 * t, 16)]
                    for j in range(16):
                        xr = iv[j]
                        row = 16 * t + j
                        for q in range(VPR):
                            buf[row * VPR + q] = (
                                ww_v[xr, pl.ds(q * 16, 16)]
                                + pos_v[row * VPR + q]
                            )

                iv_t = x_v[pl.ds(kbase + SUB - 16, 16)]
                do_group(buf, iv_t, SUB - 16, range(8, 16))

                pltpu.async_copy(
                    buf,
                    out_hbm.at[pl.ds(out_base + kk * SUB * VPR, SUB * VPR)],
                    wsems[b])
            return carry

        lax.fori_loop(0, CPT // NBUF, body, 0)
        for b in range(NBUF):
            drain_w(wsems[b], bufs[b])

    return k


@jax.jit
def kernel(x, W_words, W_pos):
    x2 = x.reshape(FLAT).astype(jnp.int32)
    wp = W_pos[:L].reshape(SUB * VPR, 16)
    out = _sc_kernel()(x2, W_words, wp)
    return out.reshape(B, L, EMBED)
